# baseline (speedup 1.0000x reference)
import dataclasses
import math

import jax
import jax.numpy as jnp
from jax import lax
from jax.experimental import pallas as pl
from jax.experimental.pallas import tpu as pltpu
from jax.experimental.pallas import tpu_sc as plsc

N_CH = 128
IN_BS = 131072
OUT_BS = math.ceil(IN_BS * 48000 / 44100)

NUM_WORKERS = 32
CW = 4464
OUT_PAD = NUM_WORKERS * CW
CW_LAST = OUT_BS - (NUM_WORKERS - 1) * CW
WB = 4160
K = 4
NT = N_CH // K
NBUF = 2


def _resample_kernel(x_hbm, fr_hbm, fl_hbm, out_hbm,
                     fl_v, fr_v, *rest):
    win = [[rest[b * K + k] for k in range(K)] for b in range(NBUF)]
    ob = [[rest[NBUF * K + b * K + k] for k in range(K)] for b in range(NBUF)]
    sem_in = rest[2 * NBUF * K:2 * NBUF * K + NBUF]
    sem_out = rest[2 * NBUF * K + NBUF:]
    wid = lax.axis_index("s") * 2 + lax.axis_index("c")
    c0 = pl.multiple_of(wid * CW, 8)

    pltpu.sync_copy(fl_hbm.at[pl.ds(c0, CW)], fl_v)
    pltpu.sync_copy(fr_hbm.at[pl.ds(c0, CW)], fr_v)

    floor0 = fl_v[pl.ds(0, 16)][0]
    start8 = pl.multiple_of(
        jnp.minimum(jnp.bitwise_and(floor0, -8), IN_BS - WB), 8)

    @plsc.parallel_loop(0, CW, step=16, unroll=4)
    def _(j):
        fl_v[pl.ds(j, 16)] = fl_v[pl.ds(j, 16)] - start8

    def fire_in(t, b):
        for k in range(K):
            off = pl.multiple_of((t * K + k) * IN_BS + start8, 8)
            pltpu.async_copy(
                x_hbm.at[pl.ds(off, WB)], win[b][k], sem_in[b])

    def wait_in(b):
        for k in range(K):
            pltpu.make_async_copy(
                x_hbm.at[pl.ds(0, WB)], win[b][k], sem_in[b]).wait()

    is_last = wid == NUM_WORKERS - 1

    def fire_out(t, b):
        for k in range(K):
            ch = t * K + k

            @pl.when(jnp.logical_not(is_last))
            def _():
                pltpu.async_copy(
                    ob[b][k],
                    out_hbm.at[pl.ds(ch, 1), pl.ds(c0, CW)], sem_out[b])

            @pl.when(is_last)
            def _():
                pltpu.async_copy(
                    ob[b][k].at[:, pl.ds(0, CW_LAST)],
                    out_hbm.at[pl.ds(ch, 1), pl.ds(c0, CW_LAST)], sem_out[b])

    def wait_out(b):
        for k in range(K):
            @pl.when(jnp.logical_not(is_last))
            def _():
                pltpu.make_async_copy(
                    ob[b][k],
                    out_hbm.at[pl.ds(0, 1), pl.ds(0, CW)], sem_out[b]).wait()

            @pl.when(is_last)
            def _():
                pltpu.make_async_copy(
                    ob[b][k].at[:, pl.ds(0, CW_LAST)],
                    out_hbm.at[pl.ds(0, 1), pl.ds(0, CW_LAST)],
                    sem_out[b]).wait()

    def compute(b):
        @plsc.parallel_loop(0, CW, step=16, unroll=4)
        def _(j):
            sl = pl.ds(j, 16)
            lv = fl_v[sl]
            l2 = jnp.minimum(lv + 1, WB - 1)
            f = fr_v[sl]
            for k in range(K):
                a = plsc.load_gather(win[b][k], [lv])
                bb = plsc.load_gather(win[b][k], [l2])
                ob[b][k][0, sl] = a + (bb - a) * f

    fire_in(0, 0)

    @pl.loop(0, NT, step=NBUF)
    def _(t):
        for b in range(NBUF):
            tt = t + b

            @pl.when(tt + 1 < NT)
            def _():
                fire_in(tt + 1, (b + 1) % NBUF)

            wait_in(b)

            @pl.when(tt >= NBUF)
            def _():
                wait_out(b)

            compute(b)
            fire_out(tt, b)

    for b in range(NBUF):
        wait_out(b)


def kernel(x, interp_in, floor_in, ceil_in):
    pad = OUT_PAD - OUT_BS
    fr = jnp.pad(interp_in, (0, pad))
    fl = jnp.pad(floor_in.astype(jnp.int32), (0, pad), constant_values=IN_BS - 1)

    cp = pltpu.CompilerParams(
        needs_layout_passes=False, use_tc_tiling_on_sc=False)
    mesh = plsc.VectorSubcoreMesh(core_axis_name="c", subcore_axis_name="s")
    run = pl.kernel(
        _resample_kernel,
        out_type=jax.ShapeDtypeStruct((N_CH, OUT_BS), jnp.float32),
        mesh=mesh,
        compiler_params=cp,
        scratch_types=[
            pltpu.VMEM((CW,), jnp.int32),
            pltpu.VMEM((CW,), jnp.float32),
            *[pltpu.VMEM((WB,), jnp.float32) for _ in range(NBUF * K)],
            *[pltpu.VMEM((1, CW), jnp.float32) for _ in range(NBUF * K)],
            *[pltpu.SemaphoreType.DMA for _ in range(2 * NBUF)],
        ],
    )
    return run(x.reshape(-1), fr, fl)


# speedup vs baseline: 2.8749x; 2.8749x over previous
import math

import jax
import jax.numpy as jnp
from jax import lax
from jax.experimental import pallas as pl
from jax.experimental.pallas import tpu as pltpu
from jax.experimental.pallas import tpu_sc as plsc

N_CH = 128
IN_BS = 131072
OUT_BS = math.ceil(IN_BS * 48000 / 44100)

NUM_WORKERS = 32
CW = 4464
OUT_PAD = NUM_WORKERS * CW
WBT = 4352
NG = N_CH // 8
NBUF = 2


def _resample_kernel(x_hbm, fr_hbm, fl_hbm, out_hbm, fl_v, fr_v, *rest):
    win = rest[:NBUF]
    ob = [rest[NBUF + 4 * h:NBUF + 4 * (h + 1)] for h in range(2)]
    sem_in = rest[NBUF + 8:NBUF + 10]
    sem_out = rest[NBUF + 10:NBUF + 12]
    wid = lax.axis_index("s") * 2 + lax.axis_index("c")
    c0 = pl.multiple_of(wid * CW, 8)

    pltpu.sync_copy(fl_hbm.at[pl.ds(c0, CW)], fl_v)
    pltpu.sync_copy(fr_hbm.at[pl.ds(c0, CW)], fr_v)

    floor0 = fl_v[pl.ds(0, 16)][0]
    col0 = pl.multiple_of(
        jnp.minimum(jnp.bitwise_and(floor0, -128), IN_BS - WBT), 128)

    @plsc.parallel_loop(0, CW, step=16, unroll=4)
    def _(j):
        fl_v[pl.ds(j, 16)] = fl_v[pl.ds(j, 16)] - col0

    def fire_in(g, b):
        pltpu.async_copy(
            x_hbm.at[pl.ds(pl.multiple_of(g * 8, 8), 8), pl.ds(col0, WBT)],
            win[b], sem_in[b])

    def wait_in(b):
        pltpu.make_async_copy(
            x_hbm.at[pl.ds(0, 8), pl.ds(0, WBT)], win[b], sem_in[b]).wait()

    def fire_out(g, h):
        for r in range(4):
            ch = g * 8 + h * 4 + r
            off = pl.multiple_of(ch * OUT_PAD + c0, 8)
            pltpu.async_copy(ob[h][r], out_hbm.at[pl.ds(off, CW)], sem_out[h])

    def wait_out(h):
        for r in range(4):
            pltpu.make_async_copy(
                ob[h][r], out_hbm.at[pl.ds(0, CW)], sem_out[h]).wait()

    def compute(b, h):
        @plsc.parallel_loop(0, CW, step=16, unroll=2)
        def _(j):
            sl = pl.ds(j, 16)
            lv = fl_v[sl]
            l2 = jnp.minimum(lv + 1, WBT - 1)
            f = fr_v[sl]
            for r in range(4):
                rv = jnp.full((16,), h * 4 + r, jnp.int32)
                a = plsc.load_gather(win[b], [rv, lv])
                bb = plsc.load_gather(win[b], [rv, l2])
                ob[h][r][sl] = a + (bb - a) * f

    fire_in(0, 0)

    @pl.loop(0, NG, step=NBUF)
    def _(g):
        for b in range(NBUF):
            gg = g + b

            @pl.when(gg + 1 < NG)
            def _():
                fire_in(gg + 1, (b + 1) % NBUF)

            wait_in(b)
            for h in range(2):
                @pl.when(gg >= 1)
                def _():
                    wait_out(h)

                compute(b, h)
                fire_out(gg, h)

    for h in range(2):
        wait_out(h)


def kernel(x, interp_in, floor_in, ceil_in):
    pad = OUT_PAD - OUT_BS
    fr = jnp.pad(interp_in, (0, pad))
    fl = jnp.pad(floor_in.astype(jnp.int32), (0, pad), constant_values=IN_BS - 1)

    cp = pltpu.CompilerParams(needs_layout_passes=False)
    mesh = plsc.VectorSubcoreMesh(core_axis_name="c", subcore_axis_name="s")
    run = pl.kernel(
        _resample_kernel,
        out_type=jax.ShapeDtypeStruct((N_CH * OUT_PAD,), jnp.float32),
        mesh=mesh,
        compiler_params=cp,
        scratch_types=[
            pltpu.VMEM((CW,), jnp.int32),
            pltpu.VMEM((CW,), jnp.float32),
            *[pltpu.VMEM((8, WBT), jnp.float32) for _ in range(NBUF)],
            *[pltpu.VMEM((CW,), jnp.float32) for _ in range(8)],
            *[pltpu.SemaphoreType.DMA for _ in range(4)],
        ],
    )
    out = run(x, fr, fl)
    return out.reshape(N_CH, OUT_PAD)[:, :OUT_BS]


# speedup vs baseline: 5.1878x; 1.8045x over previous
import math

import jax
import jax.numpy as jnp
from jax import lax
from jax.experimental import pallas as pl
from jax.experimental.pallas import tpu as pltpu
from jax.experimental.pallas import tpu_sc as plsc

N_CH = 128
IN_BS = 131072
OUT_BS = math.ceil(IN_BS * 48000 / 44100)

NUM_WORKERS = 32
CW = 4480
OUT_PAD = NUM_WORKERS * CW
CW_MAIN_LAST = OUT_BS // 128 * 128 - (NUM_WORKERS - 1) * CW
TAIL0 = OUT_BS // 128 * 128
WBT = 4352
NG = N_CH // 8
NBUF = 2


def _resample_kernel(x_hbm, fr_hbm, fl_hbm, out_hbm, tail_hbm,
                     fl_v, fr_v, *rest):
    win = rest[:NBUF]
    ob = rest[NBUF]
    sem_in = rest[NBUF + 1:NBUF + 3]
    sem_out = rest[NBUF + 3]
    wid = lax.axis_index("s") * 2 + lax.axis_index("c")
    c0 = pl.multiple_of(wid * CW, 128)
    is_last = wid == NUM_WORKERS - 1

    pltpu.sync_copy(fl_hbm.at[pl.ds(c0, CW)], fl_v)
    pltpu.sync_copy(fr_hbm.at[pl.ds(c0, CW)], fr_v)

    floor0 = fl_v[pl.ds(0, 16)][0]
    col0 = pl.multiple_of(
        jnp.minimum(jnp.bitwise_and(floor0, -128), IN_BS - WBT), 128)

    @plsc.parallel_loop(0, CW, step=16, unroll=4)
    def _(j):
        fl_v[pl.ds(j, 16)] = fl_v[pl.ds(j, 16)] - col0

    def fire_in(g, b):
        pltpu.async_copy(
            x_hbm.at[pl.ds(pl.multiple_of(g * 8, 8), 8), pl.ds(col0, WBT)],
            win[b], sem_in[b])

    def wait_in(b):
        pltpu.make_async_copy(
            x_hbm.at[pl.ds(0, 8), pl.ds(0, WBT)], win[b], sem_in[b]).wait()

    def fire_out(g):
        g8 = pl.multiple_of(g * 8, 8)

        @pl.when(jnp.logical_not(is_last))
        def _():
            pltpu.async_copy(
                ob, out_hbm.at[pl.ds(g8, 8), pl.ds(c0, CW)], sem_out)

        @pl.when(is_last)
        def _():
            pltpu.async_copy(
                ob.at[:, pl.ds(0, CW_MAIN_LAST)],
                out_hbm.at[pl.ds(g8, 8), pl.ds(c0, CW_MAIN_LAST)], sem_out)
            pltpu.async_copy(
                ob.at[:, pl.ds(CW_MAIN_LAST, 128)],
                tail_hbm.at[pl.ds(g8, 8), pl.ds(0, 128)], sem_out)

    def wait_out():
        @pl.when(jnp.logical_not(is_last))
        def _():
            pltpu.make_async_copy(
                ob, out_hbm.at[pl.ds(0, 8), pl.ds(0, CW)], sem_out).wait()

        @pl.when(is_last)
        def _():
            pltpu.make_async_copy(
                ob.at[:, pl.ds(0, CW_MAIN_LAST)],
                out_hbm.at[pl.ds(0, 8), pl.ds(0, CW_MAIN_LAST)],
                sem_out).wait()
            pltpu.make_async_copy(
                ob.at[:, pl.ds(CW_MAIN_LAST, 128)],
                tail_hbm.at[pl.ds(0, 8), pl.ds(0, 128)], sem_out).wait()

    def compute(b):
        @plsc.parallel_loop(0, CW, step=16, unroll=2)
        def _(j):
            sl = pl.ds(j, 16)
            lv = fl_v[sl]
            l2 = jnp.minimum(lv + 1, WBT - 1)
            f = fr_v[sl]
            for r in range(8):
                rv = jnp.full((16,), r, jnp.int32)
                a = plsc.load_gather(win[b], [rv, lv])
                bb = plsc.load_gather(win[b], [rv, l2])
                ob[r, sl] = a + (bb - a) * f

    fire_in(0, 0)

    @pl.loop(0, NG, step=NBUF)
    def _(g):
        for b in range(NBUF):
            gg = g + b

            @pl.when(gg + 1 < NG)
            def _():
                fire_in(gg + 1, (b + 1) % NBUF)

            wait_in(b)

            @pl.when(gg >= 1)
            def _():
                wait_out()

            compute(b)
            fire_out(gg)

    wait_out()


def kernel(x, interp_in, floor_in, ceil_in):
    pad = OUT_PAD - OUT_BS
    fr = jnp.pad(interp_in, (0, pad))
    fl = jnp.pad(floor_in.astype(jnp.int32), (0, pad), constant_values=IN_BS - 1)

    cp = pltpu.CompilerParams(needs_layout_passes=False)
    mesh = plsc.VectorSubcoreMesh(core_axis_name="c", subcore_axis_name="s")
    run = pl.kernel(
        _resample_kernel,
        out_type=(jax.ShapeDtypeStruct((N_CH, OUT_BS), jnp.float32),
                  jax.ShapeDtypeStruct((N_CH, 128), jnp.float32)),
        mesh=mesh,
        compiler_params=cp,
        scratch_types=[
            pltpu.VMEM((CW,), jnp.int32),
            pltpu.VMEM((CW,), jnp.float32),
            *[pltpu.VMEM((8, WBT), jnp.float32) for _ in range(NBUF)],
            pltpu.VMEM((8, CW), jnp.float32),
            *[pltpu.SemaphoreType.DMA for _ in range(3)],
        ],
    )
    out, tail = run(x, fr, fl)
    tail = lax.slice(tail, (0, 0), (N_CH, OUT_BS - TAIL0))
    return lax.dynamic_update_slice(out, tail, (0, TAIL0))


# speedup vs baseline: 5.2452x; 1.0111x over previous
import math

import jax
import jax.numpy as jnp
from jax import lax
from jax.experimental import pallas as pl
from jax.experimental.pallas import tpu as pltpu
from jax.experimental.pallas import tpu_sc as plsc

N_CH = 128
IN_BS = 131072
OUT_BS = math.ceil(IN_BS * 48000 / 44100)

NUM_WORKERS = 32
CW = 4480
OUT_PAD = NUM_WORKERS * CW
CW_MAIN_LAST = OUT_BS // 128 * 128 - (NUM_WORKERS - 1) * CW
TAIL0 = OUT_BS // 128 * 128
WBT = 4352
NG = N_CH // 8
NBUF = 2


def _resample_kernel(x_hbm, fr_hbm, fl_hbm, out_hbm, tail_hbm,
                     fl_v, fr_v, *rest):
    win = rest[:NBUF]
    ob = rest[NBUF]
    sem_in = rest[NBUF + 1:NBUF + 3]
    sem_out = rest[NBUF + 3]
    wid = lax.axis_index("s") * 2 + lax.axis_index("c")
    c0 = pl.multiple_of(wid * CW, 128)
    is_last = wid == NUM_WORKERS - 1

    pltpu.sync_copy(fl_hbm.at[pl.ds(c0, CW)], fl_v)
    pltpu.sync_copy(fr_hbm.at[pl.ds(c0, CW)], fr_v)

    floor0 = fl_v[pl.ds(0, 16)][0]
    col0 = pl.multiple_of(
        jnp.minimum(jnp.bitwise_and(floor0, -128), IN_BS - WBT), 128)

    @plsc.parallel_loop(0, CW, step=16, unroll=4)
    def _(j):
        fl_v[pl.ds(j, 16)] = fl_v[pl.ds(j, 16)] - col0

    def fire_in(g, b):
        pltpu.async_copy(
            x_hbm.at[pl.ds(pl.multiple_of(g * 8, 8), 8), pl.ds(col0, WBT)],
            win[b], sem_in[b])

    def wait_in(b):
        pltpu.make_async_copy(
            x_hbm.at[pl.ds(0, 8), pl.ds(0, WBT)], win[b], sem_in[b]).wait()

    def fire_out(g):
        g8 = pl.multiple_of(g * 8, 8)

        @pl.when(jnp.logical_not(is_last))
        def _():
            pltpu.async_copy(
                ob, out_hbm.at[pl.ds(g8, 8), pl.ds(c0, CW)], sem_out)

        @pl.when(is_last)
        def _():
            pltpu.async_copy(
                ob.at[:, pl.ds(0, CW_MAIN_LAST)],
                out_hbm.at[pl.ds(g8, 8), pl.ds(c0, CW_MAIN_LAST)], sem_out)
            pltpu.async_copy(
                ob.at[:, pl.ds(CW_MAIN_LAST, 128)],
                tail_hbm.at[pl.ds(g8, 8), pl.ds(0, 128)], sem_out)

    def wait_out():
        @pl.when(jnp.logical_not(is_last))
        def _():
            pltpu.make_async_copy(
                ob, out_hbm.at[pl.ds(0, 8), pl.ds(0, CW)], sem_out).wait()

        @pl.when(is_last)
        def _():
            pltpu.make_async_copy(
                ob.at[:, pl.ds(0, CW_MAIN_LAST)],
                out_hbm.at[pl.ds(0, 8), pl.ds(0, CW_MAIN_LAST)],
                sem_out).wait()
            pltpu.make_async_copy(
                ob.at[:, pl.ds(CW_MAIN_LAST, 128)],
                tail_hbm.at[pl.ds(0, 8), pl.ds(0, 128)], sem_out).wait()

    def compute(b):
        @plsc.parallel_loop(0, CW, step=16, unroll=2)
        def _(j):
            sl = pl.ds(j, 16)
            lv = fl_v[sl]
            l2 = jnp.minimum(lv + 1, WBT - 1)
            f = fr_v[sl]
            for r in range(8):
                rv = jnp.full((16,), r, jnp.int32)
                a = plsc.load_gather(win[b], [rv, lv])
                bb = plsc.load_gather(win[b], [rv, l2])
                ob[r, sl] = a + (bb - a) * f

    fire_in(0, 0)

    @pl.loop(0, NG, step=NBUF)
    def _(g):
        for b in range(NBUF):
            gg = g + b

            @pl.when(gg + 1 < NG)
            def _():
                fire_in(gg + 1, (b + 1) % NBUF)

            wait_in(b)

            @pl.when(gg >= 1)
            def _():
                wait_out()

            compute(b)
            fire_out(gg)

    wait_out()


def kernel(x, interp_in, floor_in, ceil_in):
    pad = OUT_PAD - OUT_BS
    fr = jnp.pad(interp_in, (0, pad))
    fl = jnp.pad(floor_in.astype(jnp.int32), (0, pad), constant_values=IN_BS - 1)

    cp = pltpu.CompilerParams(
        needs_layout_passes=False, use_tc_tiling_on_sc=True)
    mesh = plsc.VectorSubcoreMesh(core_axis_name="c", subcore_axis_name="s")
    run = pl.kernel(
        _resample_kernel,
        out_type=(jax.ShapeDtypeStruct((N_CH, OUT_BS), jnp.float32),
                  jax.ShapeDtypeStruct((N_CH, 128), jnp.float32)),
        mesh=mesh,
        compiler_params=cp,
        scratch_types=[
            pltpu.VMEM((CW,), jnp.int32),
            pltpu.VMEM((CW,), jnp.float32),
            *[pltpu.VMEM((8, WBT), jnp.float32) for _ in range(NBUF)],
            pltpu.VMEM((8, CW), jnp.float32),
            *[pltpu.SemaphoreType.DMA for _ in range(3)],
        ],
    )
    out, tail = run(x, fr, fl)
    tail = lax.slice(tail, (0, 0), (N_CH, OUT_BS - TAIL0))
    return lax.dynamic_update_slice(out, tail, (0, TAIL0))
